# Initial kernel scaffold; baseline (speedup 1.0000x reference)
import jax
import jax.numpy as jnp
from jax.experimental import pallas as pl


def kernel(x, edge_index, W1, b1, W2, b2, W3, b3):
    raise NotImplementedError("write your pallas kernel here")


# speedup vs baseline: 16.3512x; 16.3512x over previous
import functools

import jax
import jax.numpy as jnp
from jax import lax
from jax.experimental import pallas as pl
from jax.experimental.pallas import tpu as pltpu, tpu_sc as plsc

N_NODES = 10000
N_EDGES = 320000
D_FEAT = 128
HIDDEN = 128
N_CLASSES = 40

NC = 2
NS = 16
LANES = 16

EDGES_PER_TILE = N_EDGES // (NC * NS)
KBLK = 80
NBLK = EDGES_PER_TILE // KBLK
NB_CH = 25
NCHUNK = NBLK // NB_CH


@functools.cache
def _mesh():
    return plsc.VectorSubcoreMesh(
        core_axis_name="c", subcore_axis_name="s", num_cores=NC, num_subcores=NS
    )


def _deg_body(dst_hbm, out_hbm, dst_v, hist_v):
    c = lax.axis_index("c")
    s = lax.axis_index("s")
    w = c * NS + s
    pltpu.sync_copy(dst_hbm.at[w], dst_v)
    zeros = jnp.zeros((LANES,), jnp.float32)

    def zbody(i, _):
        hist_v[pl.ds(i * LANES, LANES)] = zeros
        return 0

    lax.fori_loop(0, N_NODES // LANES, zbody, 0)
    ones = jnp.ones((LANES,), jnp.float32)

    def body(i, _):
        idx = dst_v[pl.ds(i * LANES, LANES)]
        plsc.addupdate_scatter(hist_v, [idx], ones)
        return 0

    lax.fori_loop(0, EDGES_PER_TILE // LANES, body, 0)
    pltpu.sync_copy(hist_v, out_hbm.at[w])


@functools.cache
def _deg_call():
    return pl.kernel(
        _deg_body,
        out_type=jax.ShapeDtypeStruct((NC * NS, N_NODES), jnp.float32),
        mesh=_mesh(),
        scratch_types=[
            pltpu.VMEM((EDGES_PER_TILE,), jnp.int32),
            pltpu.VMEM((N_NODES,), jnp.float32),
        ],
        compiler_params=pltpu.CompilerParams(needs_layout_passes=False),
    )


RCHUNK = 624
RLAST = N_NODES - (NS - 1) * RCHUNK


def _stage(s, src_view, dst_view):
    r0 = pl.multiple_of(s * RCHUNK, 8)

    @pl.when(s < NS - 1)
    def _():
        pltpu.sync_copy(src_view.at[pl.ds(r0, RCHUNK)],
                        dst_view.at[pl.ds(r0, RCHUNK)])

    @pl.when(s == NS - 1)
    def _():
        pltpu.sync_copy(src_view.at[pl.ds((NS - 1) * RCHUNK, RLAST)],
                        dst_view.at[pl.ds((NS - 1) * RCHUNK, RLAST)])


def _agg_body(src_hbm, dst_hbm, y_hbm, out_hbm, src_v, dst_v, gbuf, acc_sh, gsem):
    c = lax.axis_index("c")
    s = lax.axis_index("s")
    _stage(s, y_hbm, acc_sh)
    plsc.subcore_barrier()

    def chunk(ch, _):
        pltpu.sync_copy(src_hbm.at[c, s, ch], src_v)
        pltpu.sync_copy(dst_hbm.at[c, s, ch], dst_v)

        def body(j, _):
            pltpu.async_copy(y_hbm.at[src_v.at[j]], gbuf, gsem).wait()
            pltpu.sync_copy(gbuf, acc_sh.at[dst_v.at[j]], add=True)
            return 0

        lax.fori_loop(0, NB_CH, body, 0)
        return 0

    lax.fori_loop(0, NCHUNK, chunk, 0)
    plsc.subcore_barrier()
    _stage(s, acc_sh, out_hbm.at[c])


@functools.cache
def _agg_call():
    return pl.kernel(
        _agg_body,
        out_type=jax.ShapeDtypeStruct((NC, N_NODES, D_FEAT), jnp.float32),
        mesh=_mesh(),
        scratch_types=[
            pltpu.VMEM((NB_CH, KBLK), jnp.int32),
            pltpu.VMEM((NB_CH, KBLK), jnp.int32),
            pltpu.VMEM((KBLK, D_FEAT), jnp.float32),
            pltpu.MemorySpace.VMEM_SHARED((N_NODES, D_FEAT), jnp.float32),
            pltpu.SemaphoreType.DMA,
        ],
    )


MBLK = 1000
GRID = N_NODES // MBLK


def _mm1_body(deg_ref, x_ref, w_ref, y_ref, dinv_ref):
    deg = jnp.sum(deg_ref[...], axis=0) + 1.0
    dinv = lax.rsqrt(deg)
    xw = jnp.dot(x_ref[...], w_ref[...], preferred_element_type=jnp.float32)
    y_ref[...] = xw * dinv
    dinv_ref[...] = dinv


def _mid_body(agg_ref, y_ref, dinv_ref, b1_ref, w_ref, out_ref):
    a = agg_ref[0] + agg_ref[1] - y_ref[...]
    dinv = dinv_ref[...]
    h = jnp.maximum(a * dinv + b1_ref[...], 0.0)
    out_ref[...] = jnp.dot(h, w_ref[...], preferred_element_type=jnp.float32) * dinv


def _head_body(agg_ref, y_ref, dinv_ref, b2_ref, w3_ref, b3_ref, out_ref):
    a = agg_ref[0] + agg_ref[1] - y_ref[...]
    h = jnp.maximum(a * dinv_ref[...] + b2_ref[...], 0.0)
    out_ref[...] = (
        jnp.dot(h, w3_ref[...], preferred_element_type=jnp.float32) + b3_ref[...]
    )


def _mm1(degs, x, W1):
    return pl.pallas_call(
        _mm1_body,
        grid=(GRID,),
        in_specs=[
            pl.BlockSpec((NC * NS, MBLK, 1), lambda i: (0, i, 0)),
            pl.BlockSpec((MBLK, D_FEAT), lambda i: (i, 0)),
            pl.BlockSpec((D_FEAT, HIDDEN), lambda i: (0, 0)),
        ],
        out_specs=[
            pl.BlockSpec((MBLK, HIDDEN), lambda i: (i, 0)),
            pl.BlockSpec((MBLK, 1), lambda i: (i, 0)),
        ],
        out_shape=[
            jax.ShapeDtypeStruct((N_NODES, HIDDEN), jnp.float32),
            jax.ShapeDtypeStruct((N_NODES, 1), jnp.float32),
        ],
    )(degs, x, W1)


def _mid(agg1, y1, dinv, b1, W2):
    return pl.pallas_call(
        _mid_body,
        grid=(GRID,),
        in_specs=[
            pl.BlockSpec((NC, MBLK, HIDDEN), lambda i: (0, i, 0)),
            pl.BlockSpec((MBLK, HIDDEN), lambda i: (i, 0)),
            pl.BlockSpec((MBLK, 1), lambda i: (i, 0)),
            pl.BlockSpec((1, HIDDEN), lambda i: (0, 0)),
            pl.BlockSpec((HIDDEN, HIDDEN), lambda i: (0, 0)),
        ],
        out_specs=pl.BlockSpec((MBLK, HIDDEN), lambda i: (i, 0)),
        out_shape=jax.ShapeDtypeStruct((N_NODES, HIDDEN), jnp.float32),
    )(agg1, y1, dinv, b1, W2)


def _head(agg2, y2, dinv, b2, W3, b3):
    return pl.pallas_call(
        _head_body,
        grid=(GRID,),
        in_specs=[
            pl.BlockSpec((NC, MBLK, HIDDEN), lambda i: (0, i, 0)),
            pl.BlockSpec((MBLK, HIDDEN), lambda i: (i, 0)),
            pl.BlockSpec((MBLK, 1), lambda i: (i, 0)),
            pl.BlockSpec((1, HIDDEN), lambda i: (0, 0)),
            pl.BlockSpec((HIDDEN, N_CLASSES), lambda i: (0, 0)),
            pl.BlockSpec((1, N_CLASSES), lambda i: (0, 0)),
        ],
        out_specs=pl.BlockSpec((MBLK, N_CLASSES), lambda i: (i, 0)),
        out_shape=jax.ShapeDtypeStruct((N_NODES, N_CLASSES), jnp.float32),
    )(agg2, y2, dinv, b2, W3, b3)


def kernel(x, edge_index, W1, b1, W2, b2, W3, b3):
    ei = edge_index.astype(jnp.int32)
    src = ei[0].reshape(NC, NS, NCHUNK, NB_CH, KBLK)
    dst = ei[1].reshape(NC, NS, NCHUNK, NB_CH, KBLK)
    dst_deg = ei[1].reshape(NC * NS, EDGES_PER_TILE)

    degs = _deg_call()(dst_deg)
    y1, dinv = _mm1(degs.reshape(NC * NS, N_NODES, 1), x, W1)
    agg1 = _agg_call()(src, dst, y1)
    y2 = _mid(agg1, y1, dinv, b1.reshape(1, HIDDEN), W2)
    agg2 = _agg_call()(src, dst, y2)
    logits = _head(agg2, y2, dinv, b2.reshape(1, HIDDEN), W3,
                   b3.reshape(1, N_CLASSES))
    return logits


# speedup vs baseline: 17.5543x; 1.0736x over previous
import functools

import jax
import jax.numpy as jnp
from jax import lax
from jax.experimental import pallas as pl
from jax.experimental.pallas import tpu as pltpu, tpu_sc as plsc

N_NODES = 10000
N_EDGES = 320000
D_FEAT = 128
HIDDEN = 128
N_CLASSES = 40

NC = 2
NS = 16
LANES = 16

EDGES_PER_TILE = N_EDGES // (NC * NS)
KBLK = 40
NBLK = EDGES_PER_TILE // KBLK
NB_CH = 50
NCHUNK = NBLK // NB_CH


@functools.cache
def _mesh():
    return plsc.VectorSubcoreMesh(
        core_axis_name="c", subcore_axis_name="s", num_cores=NC, num_subcores=NS
    )


def _deg_body(dst_hbm, out_hbm, dst_v, hist_v):
    c = lax.axis_index("c")
    s = lax.axis_index("s")
    w = c * NS + s
    pltpu.sync_copy(dst_hbm.at[w], dst_v)
    zeros = jnp.zeros((LANES,), jnp.float32)

    def zbody(i, _):
        hist_v[pl.ds(i * LANES, LANES)] = zeros
        return 0

    lax.fori_loop(0, N_NODES // LANES, zbody, 0)
    ones = jnp.ones((LANES,), jnp.float32)

    def body(i, _):
        idx = dst_v[pl.ds(i * LANES, LANES)]
        plsc.addupdate_scatter(hist_v, [idx], ones)
        return 0

    lax.fori_loop(0, EDGES_PER_TILE // LANES, body, 0)
    pltpu.sync_copy(hist_v, out_hbm.at[w])


@functools.cache
def _deg_call():
    return pl.kernel(
        _deg_body,
        out_type=jax.ShapeDtypeStruct((NC * NS, N_NODES), jnp.float32),
        mesh=_mesh(),
        scratch_types=[
            pltpu.VMEM((EDGES_PER_TILE,), jnp.int32),
            pltpu.VMEM((N_NODES,), jnp.float32),
        ],
        compiler_params=pltpu.CompilerParams(needs_layout_passes=False),
    )


RCHUNK = 624
RLAST = N_NODES - (NS - 1) * RCHUNK


def _stage(s, src_view, dst_view):
    r0 = pl.multiple_of(s * RCHUNK, 8)

    @pl.when(s < NS - 1)
    def _():
        pltpu.sync_copy(src_view.at[pl.ds(r0, RCHUNK)],
                        dst_view.at[pl.ds(r0, RCHUNK)])

    @pl.when(s == NS - 1)
    def _():
        pltpu.sync_copy(src_view.at[pl.ds((NS - 1) * RCHUNK, RLAST)],
                        dst_view.at[pl.ds((NS - 1) * RCHUNK, RLAST)])


def _agg_body(src_hbm, dst_hbm, y_hbm, out_hbm, src_v, dst_v, gbuf0, gbuf1,
              acc_sh, gsem0, gsem1, ssem0, ssem1):
    c = lax.axis_index("c")
    s = lax.axis_index("s")
    _stage(s, y_hbm, acc_sh)
    plsc.subcore_barrier()

    def g_start(jv, buf, sem):
        pltpu.async_copy(y_hbm.at[src_v.at[jv]], buf, sem)

    def g_wait(buf, sem):
        pltpu.make_async_copy(y_hbm.at[src_v.at[0]], buf, sem).wait()

    def s_start(jv, buf, sem):
        pltpu.async_copy(buf, acc_sh.at[dst_v.at[jv]], sem, add=True)

    def s_wait(buf, sem):
        pltpu.make_async_copy(buf, acc_sh.at[dst_v.at[0]], sem).wait()

    def chunk(ch, _):
        pltpu.sync_copy(src_hbm.at[c, s, ch], src_v)
        pltpu.sync_copy(dst_hbm.at[c, s, ch], dst_v)
        g_start(0, gbuf0, gsem0)
        g_start(1, gbuf1, gsem1)

        def pair(kk, _):
            j = 2 * kk
            g_wait(gbuf0, gsem0)
            s_start(j, gbuf0, ssem0)
            g_wait(gbuf1, gsem1)
            s_start(j + 1, gbuf1, ssem1)

            @pl.when(kk < NB_CH // 2 - 1)
            def _():
                s_wait(gbuf0, ssem0)
                g_start(j + 2, gbuf0, gsem0)
                s_wait(gbuf1, ssem1)
                g_start(j + 3, gbuf1, gsem1)

            return 0

        lax.fori_loop(0, NB_CH // 2, pair, 0)
        s_wait(gbuf0, ssem0)
        s_wait(gbuf1, ssem1)
        return 0

    lax.fori_loop(0, NCHUNK, chunk, 0)
    plsc.subcore_barrier()
    _stage(s, acc_sh, out_hbm.at[c])


@functools.cache
def _agg_call():
    return pl.kernel(
        _agg_body,
        out_type=jax.ShapeDtypeStruct((NC, N_NODES, D_FEAT), jnp.float32),
        mesh=_mesh(),
        scratch_types=[
            pltpu.VMEM((NB_CH, KBLK), jnp.int32),
            pltpu.VMEM((NB_CH, KBLK), jnp.int32),
            pltpu.VMEM((KBLK, D_FEAT), jnp.float32),
            pltpu.VMEM((KBLK, D_FEAT), jnp.float32),
            pltpu.MemorySpace.VMEM_SHARED((N_NODES, D_FEAT), jnp.float32),
            pltpu.SemaphoreType.DMA,
            pltpu.SemaphoreType.DMA,
            pltpu.SemaphoreType.DMA,
            pltpu.SemaphoreType.DMA,
        ],
    )


MBLK = 1000
GRID = N_NODES // MBLK


def _mm1_body(deg_ref, x_ref, w_ref, y_ref, dinv_ref):
    deg = jnp.sum(deg_ref[...], axis=0) + 1.0
    dinv = lax.rsqrt(deg)
    xw = jnp.dot(x_ref[...], w_ref[...], preferred_element_type=jnp.float32)
    y_ref[...] = xw * dinv
    dinv_ref[...] = dinv


def _mid_body(agg_ref, y_ref, dinv_ref, b1_ref, w_ref, out_ref):
    a = agg_ref[0] + agg_ref[1] - y_ref[...]
    dinv = dinv_ref[...]
    h = jnp.maximum(a * dinv + b1_ref[...], 0.0)
    out_ref[...] = jnp.dot(h, w_ref[...], preferred_element_type=jnp.float32) * dinv


def _head_body(agg_ref, y_ref, dinv_ref, b2_ref, w3_ref, b3_ref, out_ref):
    a = agg_ref[0] + agg_ref[1] - y_ref[...]
    h = jnp.maximum(a * dinv_ref[...] + b2_ref[...], 0.0)
    out_ref[...] = (
        jnp.dot(h, w3_ref[...], preferred_element_type=jnp.float32) + b3_ref[...]
    )


def _mm1(degs, x, W1):
    return pl.pallas_call(
        _mm1_body,
        grid=(GRID,),
        in_specs=[
            pl.BlockSpec((NC * NS, MBLK, 1), lambda i: (0, i, 0)),
            pl.BlockSpec((MBLK, D_FEAT), lambda i: (i, 0)),
            pl.BlockSpec((D_FEAT, HIDDEN), lambda i: (0, 0)),
        ],
        out_specs=[
            pl.BlockSpec((MBLK, HIDDEN), lambda i: (i, 0)),
            pl.BlockSpec((MBLK, 1), lambda i: (i, 0)),
        ],
        out_shape=[
            jax.ShapeDtypeStruct((N_NODES, HIDDEN), jnp.float32),
            jax.ShapeDtypeStruct((N_NODES, 1), jnp.float32),
        ],
    )(degs, x, W1)


def _mid(agg1, y1, dinv, b1, W2):
    return pl.pallas_call(
        _mid_body,
        grid=(GRID,),
        in_specs=[
            pl.BlockSpec((NC, MBLK, HIDDEN), lambda i: (0, i, 0)),
            pl.BlockSpec((MBLK, HIDDEN), lambda i: (i, 0)),
            pl.BlockSpec((MBLK, 1), lambda i: (i, 0)),
            pl.BlockSpec((1, HIDDEN), lambda i: (0, 0)),
            pl.BlockSpec((HIDDEN, HIDDEN), lambda i: (0, 0)),
        ],
        out_specs=pl.BlockSpec((MBLK, HIDDEN), lambda i: (i, 0)),
        out_shape=jax.ShapeDtypeStruct((N_NODES, HIDDEN), jnp.float32),
    )(agg1, y1, dinv, b1, W2)


def _head(agg2, y2, dinv, b2, W3, b3):
    return pl.pallas_call(
        _head_body,
        grid=(GRID,),
        in_specs=[
            pl.BlockSpec((NC, MBLK, HIDDEN), lambda i: (0, i, 0)),
            pl.BlockSpec((MBLK, HIDDEN), lambda i: (i, 0)),
            pl.BlockSpec((MBLK, 1), lambda i: (i, 0)),
            pl.BlockSpec((1, HIDDEN), lambda i: (0, 0)),
            pl.BlockSpec((HIDDEN, N_CLASSES), lambda i: (0, 0)),
            pl.BlockSpec((1, N_CLASSES), lambda i: (0, 0)),
        ],
        out_specs=pl.BlockSpec((MBLK, N_CLASSES), lambda i: (i, 0)),
        out_shape=jax.ShapeDtypeStruct((N_NODES, N_CLASSES), jnp.float32),
    )(agg2, y2, dinv, b2, W3, b3)


def kernel(x, edge_index, W1, b1, W2, b2, W3, b3):
    ei = edge_index.astype(jnp.int32)
    src = ei[0].reshape(NC, NS, NCHUNK, NB_CH, KBLK)
    dst = ei[1].reshape(NC, NS, NCHUNK, NB_CH, KBLK)
    dst_deg = ei[1].reshape(NC * NS, EDGES_PER_TILE)

    degs = _deg_call()(dst_deg)
    y1, dinv = _mm1(degs.reshape(NC * NS, N_NODES, 1), x, W1)
    agg1 = _agg_call()(src, dst, y1)
    y2 = _mid(agg1, y1, dinv, b1.reshape(1, HIDDEN), W2)
    agg2 = _agg_call()(src, dst, y2)
    logits = _head(agg2, y2, dinv, b2.reshape(1, HIDDEN), W3,
                   b3.reshape(1, N_CLASSES))
    return logits
